# baseline (speedup 1.0000x reference)
import functools

import jax
import jax.numpy as jnp
from jax import lax
from jax.experimental import pallas as pl
from jax.experimental.pallas import tpu as pltpu
from jax.experimental.pallas import tpu_sc as plsc

N_NODES = 16384
N_EDGES = 262144
G = 256
NPG = 64
D_STATE = 64
MSG_H = 128
HALF = MSG_H // 2
T_PROP = 3
SINK_ITERS = 20
SINK_TEMP = 0.1

NC = 2
NS = 16
KCH = 128

f32 = jnp.float32


def _enc_node_body(x_ref, wn_ref, bn_ref, w1f_ref, w1t_ref,
                   h_ref, hf_ref, ht_ref):
    h = jnp.dot(x_ref[...], wn_ref[...], preferred_element_type=f32) + bn_ref[...]
    h_ref[...] = h
    f = jnp.dot(h, w1f_ref[...], preferred_element_type=f32)
    t = jnp.dot(h, w1t_ref[...], preferred_element_type=f32)
    hf_ref[0] = f[:, :HALF]
    hf_ref[1] = f[:, HALF:]
    ht_ref[0] = t[:, :HALF]
    ht_ref[1] = t[:, HALF:]


def _enc_node(x, wn, bn, w1f, w1t):
    nb = 16
    blk = N_NODES // nb
    return pl.pallas_call(
        _enc_node_body,
        grid=(nb,),
        in_specs=[
            pl.BlockSpec((blk, 32), lambda i: (i, 0)),
            pl.BlockSpec((32, 64), lambda i: (0, 0)),
            pl.BlockSpec((1, 64), lambda i: (0, 0)),
            pl.BlockSpec((64, 128), lambda i: (0, 0)),
            pl.BlockSpec((64, 128), lambda i: (0, 0)),
        ],
        out_specs=[
            pl.BlockSpec((blk, 64), lambda i: (i, 0)),
            pl.BlockSpec((2, blk, HALF), lambda i: (0, i, 0)),
            pl.BlockSpec((2, blk, HALF), lambda i: (0, i, 0)),
        ],
        out_shape=[
            jax.ShapeDtypeStruct((N_NODES, 64), f32),
            jax.ShapeDtypeStruct((2, N_NODES, HALF), f32),
            jax.ShapeDtypeStruct((2, N_NODES, HALF), f32),
        ],
    )(x, wn, bn, w1f, w1t)


def _enc_edge_body(x_ref, we_ref, be_ref, w1e_ref, b1_ref, ec_ref):
    e = jnp.dot(x_ref[...], we_ref[...], preferred_element_type=f32) + be_ref[...]
    ec = jnp.dot(e, w1e_ref[...], preferred_element_type=f32) + b1_ref[...]
    ec_ref[0] = ec[:, :HALF]
    ec_ref[1] = ec[:, HALF:]


def _enc_edge(x, we, be, w1e, b1):
    nb = 64
    blk = N_EDGES // nb
    return pl.pallas_call(
        _enc_edge_body,
        grid=(nb,),
        in_specs=[
            pl.BlockSpec((blk, 16), lambda i: (i, 0)),
            pl.BlockSpec((16, 32), lambda i: (0, 0)),
            pl.BlockSpec((1, 32), lambda i: (0, 0)),
            pl.BlockSpec((32, 128), lambda i: (0, 0)),
            pl.BlockSpec((1, 128), lambda i: (0, 0)),
        ],
        out_specs=[pl.BlockSpec((2, blk, HALF), lambda i: (0, i, 0))],
        out_shape=[jax.ShapeDtypeStruct((2, N_EDGES, HALF), f32)],
    )(x, we, be, w1e, b1)[0]


def _update_body(h_ref, agg_ref, deg_ref, w2_ref, b2_ref, uw1_ref, ub1_ref,
                 uw2_ref, ub2_ref, w1f_ref, w1t_ref, h_out, hf_ref, ht_ref):
    h = h_ref[...]
    w2 = w2_ref[...]
    agg = (jnp.dot(agg_ref[0], w2[:HALF], preferred_element_type=f32)
           + jnp.dot(agg_ref[1], w2[HALF:], preferred_element_type=f32)
           + (deg_ref[0][:, 0:1] + deg_ref[1][:, 0:1]) * b2_ref[...])
    uw1 = uw1_ref[...]
    u = jnp.dot(h, uw1[:D_STATE], preferred_element_type=f32)
    u = u + jnp.dot(agg, uw1[D_STATE:], preferred_element_type=f32)
    u = jnp.maximum(u + ub1_ref[...], 0.0)
    hn = jnp.dot(u, uw2_ref[...], preferred_element_type=f32) + ub2_ref[...]
    h_out[...] = hn
    f = jnp.dot(hn, w1f_ref[...], preferred_element_type=f32)
    t = jnp.dot(hn, w1t_ref[...], preferred_element_type=f32)
    hf_ref[0] = f[:, :HALF]
    hf_ref[1] = f[:, HALF:]
    ht_ref[0] = t[:, :HALF]
    ht_ref[1] = t[:, HALF:]


def _update(h, agg, deg, w2, b2, uw1, ub1, uw2, ub2, w1f, w1t):
    nb = 16
    blk = N_NODES // nb
    return pl.pallas_call(
        _update_body,
        grid=(nb,),
        in_specs=[
            pl.BlockSpec((blk, 64), lambda i: (i, 0)),
            pl.BlockSpec((2, blk, HALF), lambda i: (0, i, 0)),
            pl.BlockSpec((2, blk, 16), lambda i: (0, i, 0)),
            pl.BlockSpec((128, 64), lambda i: (0, 0)),
            pl.BlockSpec((1, 64), lambda i: (0, 0)),
            pl.BlockSpec((128, 128), lambda i: (0, 0)),
            pl.BlockSpec((1, 128), lambda i: (0, 0)),
            pl.BlockSpec((128, 64), lambda i: (0, 0)),
            pl.BlockSpec((1, 64), lambda i: (0, 0)),
            pl.BlockSpec((64, 128), lambda i: (0, 0)),
            pl.BlockSpec((64, 128), lambda i: (0, 0)),
        ],
        out_specs=[
            pl.BlockSpec((blk, 64), lambda i: (i, 0)),
            pl.BlockSpec((2, blk, HALF), lambda i: (0, i, 0)),
            pl.BlockSpec((2, blk, HALF), lambda i: (0, i, 0)),
        ],
        out_shape=[
            jax.ShapeDtypeStruct((N_NODES, 64), f32),
            jax.ShapeDtypeStruct((2, N_NODES, HALF), f32),
            jax.ShapeDtypeStruct((2, N_NODES, HALF), f32),
        ],
    )(h, agg, deg, w2, b2, uw1, ub1, uw2, ub2, w1f, w1t)


def _lse(la, axis):
    m = jnp.max(la, axis=axis, keepdims=True)
    return jnp.log(jnp.sum(jnp.exp(la - m), axis=axis, keepdims=True)) + m


PAIRS_PER = 4


def _final_body(sq_ref, sc_ref, w1_ref, b1_ref, w2_ref, b2_ref, out_ref):
    w1 = w1_ref[...]
    b1 = b1_ref[...]
    w2 = w2_ref[...]
    b2 = b2_ref[...]
    vals = []
    for b in range(PAIRS_PER):
        sqb = sq_ref[b]
        scb = sc_ref[b]
        tq = jnp.dot(jnp.maximum(jnp.dot(sqb, w1, preferred_element_type=f32)
                                 + b1, 0.0), w2, preferred_element_type=f32) + b2
        tc = jnp.dot(jnp.maximum(jnp.dot(scb, w1, preferred_element_type=f32)
                                 + b1, 0.0), w2, preferred_element_type=f32) + b2
        cost = jnp.sum(jnp.abs(tq[:, None, :] - tc[None, :, :]), axis=-1)
        la = -cost / SINK_TEMP

        def body(_, la):
            la = la - _lse(la, 1)
            la = la - _lse(la, 0)
            return la

        la = lax.fori_loop(0, SINK_ITERS, body, la)
        plan = jnp.exp(la)
        align = jnp.sum(jnp.abs(sqb[:, None, :] - scb[None, :, :]), axis=-1)
        vals.append(jnp.sum(plan * align))
    v = jnp.stack(vals)
    out_ref[0] = jnp.broadcast_to(v[:, None], (PAIRS_PER, 128))


def _final(sq, sc, w1, b1, w2, b2):
    nb = (G // 2) // PAIRS_PER
    return pl.pallas_call(
        _final_body,
        grid=(nb,),
        in_specs=[
            pl.BlockSpec((PAIRS_PER, NPG, D_STATE), lambda i: (i, 0, 0)),
            pl.BlockSpec((PAIRS_PER, NPG, D_STATE), lambda i: (i, 0, 0)),
            pl.BlockSpec((64, 64), lambda i: (0, 0)),
            pl.BlockSpec((1, 64), lambda i: (0, 0)),
            pl.BlockSpec((64, 64), lambda i: (0, 0)),
            pl.BlockSpec((1, 64), lambda i: (0, 0)),
        ],
        out_specs=[pl.BlockSpec((1, PAIRS_PER, 128), lambda i: (i, 0, 0))],
        out_shape=[jax.ShapeDtypeStruct(((G // 2) // PAIRS_PER, PAIRS_PER, 128),
                                        f32)],
    )(sq, sc, w1, b1, w2, b2)[0]


EPW = N_EDGES // (NC * NS)
DCH = EPW // KCH
EPT = N_EDGES // NS
ECH = EPT // KCH
STRIPE = N_NODES // NS


def _deg_body(to_hbm, out_hbm, idx_v, ones_v, stage_v, acc_sh):
    c = lax.axis_index("c")
    s = lax.axis_index("s")
    w = s * NC + c
    base = w * EPW

    def fill(i, _):
        ones_v[i] = jnp.ones((16,), f32)
        stage_v[i] = jnp.zeros((16,), f32)
        return 0

    lax.fori_loop(0, KCH, fill, 0)

    def fill2(i, _):
        stage_v[i] = jnp.zeros((16,), f32)
        return 0

    lax.fori_loop(KCH, STRIPE, fill2, 0)
    pltpu.sync_copy(stage_v, acc_sh.at[pl.ds(s * STRIPE, STRIPE)])
    plsc.subcore_barrier()

    def chunk(j, _):
        pltpu.sync_copy(to_hbm.at[j], idx_v.at[j % DCH])
        pltpu.sync_copy(ones_v, acc_sh.at[idx_v.at[j % DCH]], add=True)
        return 0

    lax.fori_loop(base // KCH, base // KCH + DCH, chunk, 0)
    plsc.subcore_barrier()
    pltpu.sync_copy(acc_sh.at[pl.ds(s * STRIPE, STRIPE)], stage_v)
    pltpu.sync_copy(stage_v, out_hbm.at[c].at[pl.ds(s * STRIPE, STRIPE)])


def _edge_body(hf_hbm, ht_hbm, ec_hbm, from2_hbm, to2_hbm, out_hbm,
               idxf_v, idxt_v, gf_v, gt_v, ecb_v, zb_v, sem1, sem2, acc_sh):
    c = lax.axis_index("c")
    s = lax.axis_index("s")

    pltpu.sync_copy(from2_hbm.at[pl.ds(s * ECH, ECH)], idxf_v)
    pltpu.sync_copy(to2_hbm.at[pl.ds(s * ECH, ECH)], idxt_v)

    def zfill(i, _):
        for g in range(HALF // 16):
            zb_v[i, pl.ds(g * 16, 16)] = jnp.zeros((16,), f32)
        return 0

    lax.fori_loop(0, KCH, zfill, 0)
    for q in range(STRIPE // KCH):
        pltpu.sync_copy(zb_v, acc_sh.at[pl.ds(s * STRIPE + q * KCH, KCH)])
    plsc.subcore_barrier()

    def chunk(j, _):
        cp1 = pltpu.async_copy(hf_hbm.at[c].at[idxf_v.at[j]], gf_v, sem1)
        cp2 = pltpu.async_copy(ht_hbm.at[c].at[idxt_v.at[j]], gt_v, sem2)
        pltpu.sync_copy(ec_hbm.at[c].at[pl.ds(s * EPT + j * KCH, KCH)], ecb_v)
        cp1.wait()
        cp2.wait()

        def compute(k, _):
            for g in range(HALF // 16):
                d = pl.ds(g * 16, 16)
                zb_v[k, d] = jnp.maximum(gf_v[k, d] + gt_v[k, d] + ecb_v[k, d],
                                         0.0)
            return 0

        lax.fori_loop(0, KCH, compute, 0)
        pltpu.sync_copy(zb_v, acc_sh.at[idxt_v.at[j]], add=True)
        return 0

    lax.fori_loop(0, ECH, chunk, 0)
    plsc.subcore_barrier()

    for q in range(STRIPE // KCH):
        r = pl.ds(s * STRIPE + q * KCH, KCH)
        pltpu.sync_copy(acc_sh.at[r], gf_v)
        pltpu.sync_copy(gf_v, out_hbm.at[c].at[r])


_SC_CACHE = {}


def _sc_kernels():
    if not _SC_CACHE:
        mesh = plsc.VectorSubcoreMesh(core_axis_name="c", subcore_axis_name="s",
                                      num_cores=NC, num_subcores=NS)
        params = pltpu.CompilerParams(use_tc_tiling_on_sc=False)
        _SC_CACHE["deg"] = pl.kernel(
            _deg_body,
            out_type=jax.ShapeDtypeStruct((NC, N_NODES, 16), f32),
            mesh=mesh,
            compiler_params=params,
            scratch_types=[
                pltpu.VMEM((DCH, KCH), jnp.int32),
                pltpu.VMEM((KCH, 16), f32),
                pltpu.VMEM((STRIPE, 16), f32),
                pltpu.VMEM_SHARED((N_NODES, 16), f32),
            ],
        )
        _SC_CACHE["edge"] = pl.kernel(
            _edge_body,
            out_type=jax.ShapeDtypeStruct((NC, N_NODES, HALF), f32),
            mesh=mesh,
            compiler_params=params,
            scratch_types=[
                pltpu.VMEM((ECH, KCH), jnp.int32),
                pltpu.VMEM((ECH, KCH), jnp.int32),
                pltpu.VMEM((KCH, HALF), f32),
                pltpu.VMEM((KCH, HALF), f32),
                pltpu.VMEM((KCH, HALF), f32),
                pltpu.VMEM((KCH, HALF), f32),
                pltpu.SemaphoreType.DMA,
                pltpu.SemaphoreType.DMA,
                pltpu.VMEM_SHARED((N_NODES, HALF), f32),
            ],
        )
    return _SC_CACHE


def _deg_call(to2):
    return _sc_kernels()["deg"](to2)


def _edge_call(hf, ht, ec, from2, to2):
    return _sc_kernels()["edge"](hf, ht, ec, from2, to2)


def kernel(node_features, edge_features, from_idx, to_idx, graph_idx,
           graph_sizes, query_adj, corpus_adj,
           enc_node_W, enc_node_b, enc_edge_W, enc_edge_b,
           msg_W1, msg_b1, msg_W2, msg_b2,
           upd_W1, upd_b1, upd_W2, upd_b2,
           sink_W1, sink_b1, sink_W2, sink_b2):
    w1f = msg_W1[0:D_STATE]
    w1t = msg_W1[D_STATE:2 * D_STATE]
    w1e = msg_W1[2 * D_STATE:]
    bn = enc_node_b.reshape(1, -1)
    be = enc_edge_b.reshape(1, -1)
    b1 = msg_b1.reshape(1, -1)
    b2 = msg_b2.reshape(1, -1)
    ub1 = upd_b1.reshape(1, -1)
    ub2 = upd_b2.reshape(1, -1)
    sb1 = sink_b1.reshape(1, -1)
    sb2 = sink_b2.reshape(1, -1)

    h, hf, ht = _enc_node(node_features, enc_node_W, bn, w1f, w1t)
    ec = _enc_edge(edge_features, enc_edge_W, be, w1e, b1)
    from2 = from_idx.reshape(N_EDGES // KCH, KCH)
    to2 = to_idx.reshape(N_EDGES // KCH, KCH)
    deg = _deg_call(to2)

    for _ in range(T_PROP):
        agg = _edge_call(hf, ht, ec, from2, to2)
        h, hf, ht = _update(h, agg, deg, msg_W2, b2, upd_W1, ub1,
                            upd_W2, ub2, w1f, w1t)

    stacked = h.reshape(G, NPG, D_STATE)
    out = _final(stacked[0::2], stacked[1::2], sink_W1, sb1, sink_W2, sb2)
    return out[:, :, 0].reshape(G // 2)


# speedup vs baseline: 4.3441x; 4.3441x over previous
import functools

import jax
import jax.numpy as jnp
from jax import lax
from jax.experimental import pallas as pl
from jax.experimental.pallas import tpu as pltpu
from jax.experimental.pallas import tpu_sc as plsc

N_NODES = 16384
N_EDGES = 262144
G = 256
NPG = 64
D_STATE = 64
MSG_H = 128
HALF = MSG_H // 2
T_PROP = 3
SINK_ITERS = 20
SINK_TEMP = 0.1

NC = 2
NS = 16
KCH = 128

f32 = jnp.float32


def _enc_node_body(x_ref, wn_ref, bn_ref, w1f_ref, w1t_ref,
                   h_ref, hf_ref, ht_ref):
    h = jnp.dot(x_ref[...], wn_ref[...], preferred_element_type=f32) + bn_ref[...]
    h_ref[...] = h
    f = jnp.dot(h, w1f_ref[...], preferred_element_type=f32)
    t = jnp.dot(h, w1t_ref[...], preferred_element_type=f32)
    hf_ref[0] = f[:, :HALF]
    hf_ref[1] = f[:, HALF:]
    ht_ref[0] = t[:, :HALF]
    ht_ref[1] = t[:, HALF:]


def _enc_node(x, wn, bn, w1f, w1t):
    nb = 16
    blk = N_NODES // nb
    return pl.pallas_call(
        _enc_node_body,
        grid=(nb,),
        in_specs=[
            pl.BlockSpec((blk, 32), lambda i: (i, 0)),
            pl.BlockSpec((32, 64), lambda i: (0, 0)),
            pl.BlockSpec((1, 64), lambda i: (0, 0)),
            pl.BlockSpec((64, 128), lambda i: (0, 0)),
            pl.BlockSpec((64, 128), lambda i: (0, 0)),
        ],
        out_specs=[
            pl.BlockSpec((blk, 64), lambda i: (i, 0)),
            pl.BlockSpec((2, blk, HALF), lambda i: (0, i, 0)),
            pl.BlockSpec((2, blk, HALF), lambda i: (0, i, 0)),
        ],
        out_shape=[
            jax.ShapeDtypeStruct((N_NODES, 64), f32),
            jax.ShapeDtypeStruct((2, N_NODES, HALF), f32),
            jax.ShapeDtypeStruct((2, N_NODES, HALF), f32),
        ],
    )(x, wn, bn, w1f, w1t)


def _enc_edge_body(x_ref, we_ref, be_ref, w1e_ref, b1_ref, ec_ref):
    e = jnp.dot(x_ref[...], we_ref[...], preferred_element_type=f32) + be_ref[...]
    ec = jnp.dot(e, w1e_ref[...], preferred_element_type=f32) + b1_ref[...]
    ec_ref[0] = ec[:, :HALF]
    ec_ref[1] = ec[:, HALF:]


def _enc_edge(x, we, be, w1e, b1):
    nb = 64
    blk = N_EDGES // nb
    return pl.pallas_call(
        _enc_edge_body,
        grid=(nb,),
        in_specs=[
            pl.BlockSpec((blk, 16), lambda i: (i, 0)),
            pl.BlockSpec((16, 32), lambda i: (0, 0)),
            pl.BlockSpec((1, 32), lambda i: (0, 0)),
            pl.BlockSpec((32, 128), lambda i: (0, 0)),
            pl.BlockSpec((1, 128), lambda i: (0, 0)),
        ],
        out_specs=[pl.BlockSpec((2, blk, HALF), lambda i: (0, i, 0))],
        out_shape=[jax.ShapeDtypeStruct((2, N_EDGES, HALF), f32)],
    )(x, we, be, w1e, b1)[0]


def _update_body(h_ref, agg_ref, deg_ref, w2_ref, b2_ref, uw1_ref, ub1_ref,
                 uw2_ref, ub2_ref, w1f_ref, w1t_ref, h_out, hf_ref, ht_ref):
    h = h_ref[...]
    w2 = w2_ref[...]
    agg = (jnp.dot(agg_ref[0], w2[:HALF], preferred_element_type=f32)
           + jnp.dot(agg_ref[1], w2[HALF:], preferred_element_type=f32)
           + (deg_ref[0][:, 0:1] + deg_ref[1][:, 0:1]) * b2_ref[...])
    uw1 = uw1_ref[...]
    u = jnp.dot(h, uw1[:D_STATE], preferred_element_type=f32)
    u = u + jnp.dot(agg, uw1[D_STATE:], preferred_element_type=f32)
    u = jnp.maximum(u + ub1_ref[...], 0.0)
    hn = jnp.dot(u, uw2_ref[...], preferred_element_type=f32) + ub2_ref[...]
    h_out[...] = hn
    f = jnp.dot(hn, w1f_ref[...], preferred_element_type=f32)
    t = jnp.dot(hn, w1t_ref[...], preferred_element_type=f32)
    hf_ref[0] = f[:, :HALF]
    hf_ref[1] = f[:, HALF:]
    ht_ref[0] = t[:, :HALF]
    ht_ref[1] = t[:, HALF:]


def _update(h, agg, deg, w2, b2, uw1, ub1, uw2, ub2, w1f, w1t):
    nb = 16
    blk = N_NODES // nb
    return pl.pallas_call(
        _update_body,
        grid=(nb,),
        in_specs=[
            pl.BlockSpec((blk, 64), lambda i: (i, 0)),
            pl.BlockSpec((2, blk, HALF), lambda i: (0, i, 0)),
            pl.BlockSpec((2, blk, 16), lambda i: (0, i, 0)),
            pl.BlockSpec((128, 64), lambda i: (0, 0)),
            pl.BlockSpec((1, 64), lambda i: (0, 0)),
            pl.BlockSpec((128, 128), lambda i: (0, 0)),
            pl.BlockSpec((1, 128), lambda i: (0, 0)),
            pl.BlockSpec((128, 64), lambda i: (0, 0)),
            pl.BlockSpec((1, 64), lambda i: (0, 0)),
            pl.BlockSpec((64, 128), lambda i: (0, 0)),
            pl.BlockSpec((64, 128), lambda i: (0, 0)),
        ],
        out_specs=[
            pl.BlockSpec((blk, 64), lambda i: (i, 0)),
            pl.BlockSpec((2, blk, HALF), lambda i: (0, i, 0)),
            pl.BlockSpec((2, blk, HALF), lambda i: (0, i, 0)),
        ],
        out_shape=[
            jax.ShapeDtypeStruct((N_NODES, 64), f32),
            jax.ShapeDtypeStruct((2, N_NODES, HALF), f32),
            jax.ShapeDtypeStruct((2, N_NODES, HALF), f32),
        ],
    )(h, agg, deg, w2, b2, uw1, ub1, uw2, ub2, w1f, w1t)


def _lse(la, axis):
    m = jnp.max(la, axis=axis, keepdims=True)
    return jnp.log(jnp.sum(jnp.exp(la - m), axis=axis, keepdims=True)) + m


PAIRS_PER = 16


def _cdist_l1(a, b):
    cols = []
    for j in range(NPG):
        cols.append(jnp.sum(jnp.abs(a - b[:, j:j + 1, :]), axis=-1))
    return jnp.stack(cols, axis=-1)


def _final_body(sq_ref, sc_ref, w1_ref, b1_ref, w2_ref, b2_ref, out_ref):
    w1 = w1_ref[...]
    b1 = b1_ref[...]
    w2 = w2_ref[...]
    b2 = b2_ref[...]
    sq = sq_ref[...]
    sc = sc_ref[...]

    def mlp(x):
        x2 = x.reshape(PAIRS_PER * NPG, D_STATE)
        y = jnp.maximum(jnp.dot(x2, w1, preferred_element_type=f32) + b1, 0.0)
        y = jnp.dot(y, w2, preferred_element_type=f32) + b2
        return y.reshape(PAIRS_PER, NPG, NPG)

    tq = mlp(sq)
    tc = mlp(sc)
    cost = _cdist_l1(tq, tc)
    la = -cost / SINK_TEMP

    def body(_, la):
        la = la - _lse(la, 2)
        la = la - _lse(la, 1)
        return la

    la = lax.fori_loop(0, SINK_ITERS, body, la)
    plan = jnp.exp(la)
    align = _cdist_l1(sq, sc)
    v = jnp.sum(plan * align, axis=(1, 2))
    out_ref[0] = jnp.broadcast_to(v[:, None], (PAIRS_PER, 128))


def _final(sq, sc, w1, b1, w2, b2):
    nb = (G // 2) // PAIRS_PER
    return pl.pallas_call(
        _final_body,
        grid=(nb,),
        in_specs=[
            pl.BlockSpec((PAIRS_PER, NPG, D_STATE), lambda i: (i, 0, 0)),
            pl.BlockSpec((PAIRS_PER, NPG, D_STATE), lambda i: (i, 0, 0)),
            pl.BlockSpec((64, 64), lambda i: (0, 0)),
            pl.BlockSpec((1, 64), lambda i: (0, 0)),
            pl.BlockSpec((64, 64), lambda i: (0, 0)),
            pl.BlockSpec((1, 64), lambda i: (0, 0)),
        ],
        out_specs=[pl.BlockSpec((1, PAIRS_PER, 128), lambda i: (i, 0, 0))],
        out_shape=[jax.ShapeDtypeStruct(((G // 2) // PAIRS_PER, PAIRS_PER, 128),
                                        f32)],
    )(sq, sc, w1, b1, w2, b2)[0]


EPW = N_EDGES // (NC * NS)
DCH = EPW // KCH
EPT = N_EDGES // NS
ECH = EPT // KCH
STRIPE = N_NODES // NS


def _deg_body(to_hbm, out_hbm, idx_v, ones_v, stage_v, acc_sh):
    c = lax.axis_index("c")
    s = lax.axis_index("s")
    w = s * NC + c
    base = w * EPW

    def fill(i, _):
        ones_v[i] = jnp.ones((16,), f32)
        stage_v[i] = jnp.zeros((16,), f32)
        return 0

    lax.fori_loop(0, KCH, fill, 0)

    def fill2(i, _):
        stage_v[i] = jnp.zeros((16,), f32)
        return 0

    lax.fori_loop(KCH, STRIPE, fill2, 0)
    pltpu.sync_copy(stage_v, acc_sh.at[pl.ds(s * STRIPE, STRIPE)])
    plsc.subcore_barrier()

    def chunk(j, _):
        pltpu.sync_copy(to_hbm.at[j], idx_v.at[j % DCH])
        pltpu.sync_copy(ones_v, acc_sh.at[idx_v.at[j % DCH]], add=True)
        return 0

    lax.fori_loop(base // KCH, base // KCH + DCH, chunk, 0)
    plsc.subcore_barrier()
    pltpu.sync_copy(acc_sh.at[pl.ds(s * STRIPE, STRIPE)], stage_v)
    pltpu.sync_copy(stage_v, out_hbm.at[c].at[pl.ds(s * STRIPE, STRIPE)])


def _edge_body(hf_hbm, ht_hbm, ec_hbm, from2_hbm, to2_hbm, out_hbm,
               idxf_v, idxt_v, gf_v, gt_v, ecb_v, zb_v, sem1, sem2, acc_sh):
    c = lax.axis_index("c")
    s = lax.axis_index("s")

    pltpu.sync_copy(from2_hbm.at[pl.ds(s * ECH, ECH)], idxf_v)
    pltpu.sync_copy(to2_hbm.at[pl.ds(s * ECH, ECH)], idxt_v)

    def zfill(i, _):
        for g in range(HALF // 16):
            zb_v[i, pl.ds(g * 16, 16)] = jnp.zeros((16,), f32)
        return 0

    lax.fori_loop(0, KCH, zfill, 0)
    for q in range(STRIPE // KCH):
        pltpu.sync_copy(zb_v, acc_sh.at[pl.ds(s * STRIPE + q * KCH, KCH)])
    plsc.subcore_barrier()

    def chunk(j, _):
        cp1 = pltpu.async_copy(hf_hbm.at[c].at[idxf_v.at[j]], gf_v, sem1)
        cp2 = pltpu.async_copy(ht_hbm.at[c].at[idxt_v.at[j]], gt_v, sem2)
        pltpu.sync_copy(ec_hbm.at[c].at[pl.ds(s * EPT + j * KCH, KCH)], ecb_v)
        cp1.wait()
        cp2.wait()

        def compute(k, _):
            for g in range(HALF // 16):
                d = pl.ds(g * 16, 16)
                zb_v[k, d] = jnp.maximum(gf_v[k, d] + gt_v[k, d] + ecb_v[k, d],
                                         0.0)
            return 0

        lax.fori_loop(0, KCH, compute, 0)
        pltpu.sync_copy(zb_v, acc_sh.at[idxt_v.at[j]], add=True)
        return 0

    lax.fori_loop(0, ECH, chunk, 0)
    plsc.subcore_barrier()

    for q in range(STRIPE // KCH):
        r = pl.ds(s * STRIPE + q * KCH, KCH)
        pltpu.sync_copy(acc_sh.at[r], gf_v)
        pltpu.sync_copy(gf_v, out_hbm.at[c].at[r])


_SC_CACHE = {}


def _sc_kernels():
    if not _SC_CACHE:
        mesh = plsc.VectorSubcoreMesh(core_axis_name="c", subcore_axis_name="s",
                                      num_cores=NC, num_subcores=NS)
        params = pltpu.CompilerParams(use_tc_tiling_on_sc=False)
        _SC_CACHE["deg"] = pl.kernel(
            _deg_body,
            out_type=jax.ShapeDtypeStruct((NC, N_NODES, 16), f32),
            mesh=mesh,
            compiler_params=params,
            scratch_types=[
                pltpu.VMEM((DCH, KCH), jnp.int32),
                pltpu.VMEM((KCH, 16), f32),
                pltpu.VMEM((STRIPE, 16), f32),
                pltpu.VMEM_SHARED((N_NODES, 16), f32),
            ],
        )
        _SC_CACHE["edge"] = pl.kernel(
            _edge_body,
            out_type=jax.ShapeDtypeStruct((NC, N_NODES, HALF), f32),
            mesh=mesh,
            compiler_params=params,
            scratch_types=[
                pltpu.VMEM((ECH, KCH), jnp.int32),
                pltpu.VMEM((ECH, KCH), jnp.int32),
                pltpu.VMEM((KCH, HALF), f32),
                pltpu.VMEM((KCH, HALF), f32),
                pltpu.VMEM((KCH, HALF), f32),
                pltpu.VMEM((KCH, HALF), f32),
                pltpu.SemaphoreType.DMA,
                pltpu.SemaphoreType.DMA,
                pltpu.VMEM_SHARED((N_NODES, HALF), f32),
            ],
        )
    return _SC_CACHE


def _deg_call(to2):
    return _sc_kernels()["deg"](to2)


def _edge_call(hf, ht, ec, from2, to2):
    return _sc_kernels()["edge"](hf, ht, ec, from2, to2)


def kernel(node_features, edge_features, from_idx, to_idx, graph_idx,
           graph_sizes, query_adj, corpus_adj,
           enc_node_W, enc_node_b, enc_edge_W, enc_edge_b,
           msg_W1, msg_b1, msg_W2, msg_b2,
           upd_W1, upd_b1, upd_W2, upd_b2,
           sink_W1, sink_b1, sink_W2, sink_b2):
    w1f = msg_W1[0:D_STATE]
    w1t = msg_W1[D_STATE:2 * D_STATE]
    w1e = msg_W1[2 * D_STATE:]
    bn = enc_node_b.reshape(1, -1)
    be = enc_edge_b.reshape(1, -1)
    b1 = msg_b1.reshape(1, -1)
    b2 = msg_b2.reshape(1, -1)
    ub1 = upd_b1.reshape(1, -1)
    ub2 = upd_b2.reshape(1, -1)
    sb1 = sink_b1.reshape(1, -1)
    sb2 = sink_b2.reshape(1, -1)

    h, hf, ht = _enc_node(node_features, enc_node_W, bn, w1f, w1t)
    ec = _enc_edge(edge_features, enc_edge_W, be, w1e, b1)
    from2 = from_idx.reshape(N_EDGES // KCH, KCH)
    to2 = to_idx.reshape(N_EDGES // KCH, KCH)
    deg = _deg_call(to2)

    for _ in range(T_PROP):
        agg = _edge_call(hf, ht, ec, from2, to2)
        h, hf, ht = _update(h, agg, deg, msg_W2, b2, upd_W1, ub1,
                            upd_W2, ub2, w1f, w1t)

    stacked = h.reshape(G, NPG, D_STATE)
    out = _final(stacked[0::2], stacked[1::2], sink_W1, sb1, sink_W2, sb2)
    return out[:, :, 0].reshape(G // 2)
